# Initial kernel scaffold; baseline (speedup 1.0000x reference)
import jax
import jax.numpy as jnp
from jax.experimental import pallas as pl


def kernel(x, W, b, gW, nW):
    raise NotImplementedError("write your pallas kernel here")


# speedup vs baseline: 3.7939x; 3.7939x over previous
import jax
import jax.numpy as jnp
from jax.experimental import pallas as pl
from jax.experimental.pallas import tpu as pltpu

_LAYERS = 4
_E = 8


def _crossnet_block(x_ref, noise_ref, cw_ref, b_ref, o_ref):
    x0 = x_ref[:]
    xl = x0
    neg_inf = jnp.float32(-jnp.inf)
    for l in range(_LAYERS):
        m = jnp.dot(xl, cw_ref[:, l * 3 * _E:(l + 1) * 3 * _E],
                    preferred_element_type=jnp.float32)
        gate_out = m[:, 0:_E]
        sp = jax.nn.softplus(m[:, _E:2 * _E])
        lin = m[:, 2 * _E:3 * _E] + b_ref[0:1, l * _E:(l + 1) * _E]
        s = gate_out + noise_ref[:, l * _E:(l + 1) * _E] * sp
        m1 = jnp.max(s, axis=1, keepdims=True)
        idx = jax.lax.broadcasted_iota(jnp.int32, s.shape, 1)
        first_idx = jnp.min(jnp.where(s == m1, idx, _E), axis=1, keepdims=True)
        m2 = jnp.max(jnp.where(idx == first_idx, neg_inf, s), axis=1,
                     keepdims=True)
        sm = jnp.where(s < m2, neg_inf, s)
        ex = jnp.exp(sm - m1)
        gate = ex / jnp.sum(ex, axis=1, keepdims=True)
        coef = jnp.sum(gate * lin, axis=1, keepdims=True)
        xl = xl + coef * x0
    o_ref[:] = xl


def kernel(x, W, b, gW, nW):
    N, D = x.shape
    nkey = jax.random.key(42)
    noise = jnp.concatenate(
        [jax.random.normal(jax.random.fold_in(nkey, l), (N, _E), dtype=jnp.float32)
         for l in range(_LAYERS)], axis=1)
    cw = jnp.concatenate(
        [jnp.concatenate([gW.T, nW.T, W[l].T], axis=1) for l in range(_LAYERS)],
        axis=1)
    b2 = jnp.tile(b.reshape(1, _LAYERS * _E), (8, 1))

    T = 512
    out = pl.pallas_call(
        _crossnet_block,
        grid=(N // T,),
        in_specs=[
            pl.BlockSpec((T, D), lambda i: (i, 0)),
            pl.BlockSpec((T, _LAYERS * _E), lambda i: (i, 0)),
            pl.BlockSpec((D, _LAYERS * 3 * _E), lambda i: (0, 0)),
            pl.BlockSpec((8, _LAYERS * _E), lambda i: (0, 0)),
        ],
        out_specs=pl.BlockSpec((T, D), lambda i: (i, 0)),
        out_shape=jax.ShapeDtypeStruct((N, D), jnp.float32),
        compiler_params=pltpu.CompilerParams(
            dimension_semantics=("arbitrary",)),
    )(x, noise, cw, b2)
    return out


# speedup vs baseline: 10.0459x; 2.6479x over previous
import jax
import jax.numpy as jnp
from jax.experimental import pallas as pl
from jax.experimental.pallas import tpu as pltpu

_LAYERS = 4
_E = 8
_S = 32


def _crossnet_block(x_ref, noise_ref, cw_ref, b_ref, o_ref):
    x0 = x_ref[:]
    xl = x0
    neg_inf = jnp.float32(-jnp.inf)
    for l in range(_LAYERS):
        mt = jax.lax.dot_general(
            cw_ref[l * _S:(l + 1) * _S, :], xl,
            (((1,), (1,)), ((), ())),
            preferred_element_type=jnp.float32)
        gate_out = mt[0:_E, :]
        sp = jax.nn.softplus(mt[_E:2 * _E, :])
        lin = mt[2 * _E:3 * _E, :] + b_ref[l * _E:(l + 1) * _E, 0:1]
        s = gate_out + noise_ref[l * _E:(l + 1) * _E, :] * sp
        m1 = jnp.max(s, axis=0, keepdims=True)
        idx = jax.lax.broadcasted_iota(jnp.int32, s.shape, 0)
        first_idx = jnp.min(jnp.where(s == m1, idx, _E), axis=0, keepdims=True)
        m2 = jnp.max(jnp.where(idx == first_idx, neg_inf, s), axis=0,
                     keepdims=True)
        sm = jnp.where(s < m2, neg_inf, s)
        ex = jnp.exp(sm - m1)
        gate = ex / jnp.sum(ex, axis=0, keepdims=True)
        coef_t = jnp.sum(gate * lin, axis=0, keepdims=True)
        coef = coef_t.T
        xl = xl + coef * x0
    o_ref[:] = xl


def kernel(x, W, b, gW, nW):
    N, D = x.shape
    nkey = jax.random.key(42)
    noise_t = jnp.concatenate(
        [jax.random.normal(jax.random.fold_in(nkey, l), (N, _E), dtype=jnp.float32).T
         for l in range(_LAYERS)], axis=0)
    pad = jnp.zeros((_S - 3 * _E, W.shape[2]), dtype=jnp.float32)
    cw = jnp.concatenate(
        sum([[gW, nW, W[l], pad] for l in range(_LAYERS)], []), axis=0)
    bt = jnp.tile(b.reshape(_LAYERS * _E, 1), (1, 128))

    T = 512
    out = pl.pallas_call(
        _crossnet_block,
        grid=(N // T,),
        in_specs=[
            pl.BlockSpec((T, D), lambda i: (i, 0)),
            pl.BlockSpec((_LAYERS * _E, T), lambda i: (0, i)),
            pl.BlockSpec((_LAYERS * _S, D), lambda i: (0, 0)),
            pl.BlockSpec((_LAYERS * _E, 128), lambda i: (0, 0)),
        ],
        out_specs=pl.BlockSpec((T, D), lambda i: (i, 0)),
        out_shape=jax.ShapeDtypeStruct((N, D), jnp.float32),
        compiler_params=pltpu.CompilerParams(
            dimension_semantics=("arbitrary",)),
    )(x, noise_t, cw, bt)
    return out


# speedup vs baseline: 14.8808x; 1.4813x over previous
import jax
import jax.numpy as jnp
from jax.experimental import pallas as pl
from jax.experimental.pallas import tpu as pltpu

_LAYERS = 4
_E = 8
_R = (2 + _LAYERS) * _E


def _crossnet_block(x_ref, noise_ref, cw_ref, b_ref, o_ref):
    x0 = x_ref[:]
    mt = jax.lax.dot_general(
        cw_ref[:], x0, (((1,), (1,)), ((), ())),
        preferred_element_type=jnp.float32)
    g0 = mt[0:_E, :]
    n0 = mt[_E:2 * _E, :]
    neg_inf = jnp.float32(-jnp.inf)
    idx = jax.lax.broadcasted_iota(jnp.int32, g0.shape, 0)
    a = jnp.ones_like(mt[0:1, :])
    for l in range(_LAYERS):
        gate_out = a * g0
        sp = jax.nn.softplus(a * n0)
        s = gate_out + noise_ref[l * _E:(l + 1) * _E, :] * sp
        m1 = jnp.max(s, axis=0, keepdims=True)
        first_idx = jnp.min(jnp.where(s == m1, idx, _E), axis=0, keepdims=True)
        m2 = jnp.max(jnp.where(idx == first_idx, neg_inf, s), axis=0,
                     keepdims=True)
        sm = jnp.where(s < m2, neg_inf, s)
        ex = jnp.exp(sm - m1)
        gate = ex / jnp.sum(ex, axis=0, keepdims=True)
        lin = a * mt[(2 + l) * _E:(3 + l) * _E, :] \
            + b_ref[l * _E:(l + 1) * _E, 0:1]
        a = a + jnp.sum(gate * lin, axis=0, keepdims=True)
    o_ref[:] = x0 * a.T


def kernel(x, W, b, gW, nW):
    N, D = x.shape
    nkey = jax.random.key(42)
    noise_t = jnp.concatenate(
        [jax.random.normal(jax.random.fold_in(nkey, l), (N, _E), dtype=jnp.float32).T
         for l in range(_LAYERS)], axis=0)
    cw = jnp.concatenate([gW, nW] + [W[l] for l in range(_LAYERS)], axis=0)
    bt = jnp.tile(b.reshape(_LAYERS * _E, 1), (1, 128))

    T = 512
    out = pl.pallas_call(
        _crossnet_block,
        grid=(N // T,),
        in_specs=[
            pl.BlockSpec((T, D), lambda i: (i, 0)),
            pl.BlockSpec((_LAYERS * _E, T), lambda i: (0, i)),
            pl.BlockSpec((_R, D), lambda i: (0, 0)),
            pl.BlockSpec((_LAYERS * _E, 128), lambda i: (0, 0)),
        ],
        out_specs=pl.BlockSpec((T, D), lambda i: (i, 0)),
        out_shape=jax.ShapeDtypeStruct((N, D), jnp.float32),
        compiler_params=pltpu.CompilerParams(
            dimension_semantics=("arbitrary",)),
    )(x, noise_t, cw, bt)
    return out
